# Initial kernel scaffold; baseline (speedup 1.0000x reference)
import jax
import jax.numpy as jnp
from jax.experimental import pallas as pl


def kernel(word_data, char_data, emb0, emb1, emb0_char):
    raise NotImplementedError("write your pallas kernel here")


# speedup vs baseline: 1.7091x; 1.7091x over previous
import functools

import jax
import jax.numpy as jnp
from jax import lax
from jax.experimental import pallas as pl
from jax.experimental.pallas import tpu as pltpu
from jax.experimental.pallas import tpu_sc as plsc

VOCAB = 1000000
CHAR_VOCAB = 20000
DIM = 64
B = 16384
NEG = 5
MAXWL = 8

NC = 2
NS = 16
NW = NC * NS
ROWS_PER_W = B // NW
CHUNK = 64
NCHUNK = ROWS_PER_W // CHUNK
LANES = 16
KV = DIM // LANES


def _sc_body(tar_idx, ctx_idx, neg_idx, char_idx, char_num,
             emb0, emb1, emb0c,
             pos_out, neg_out,
             tar_i, ctx_i, neg_i, char_i, num_v,
             tar_r, ctx_r, neg_r, char_r,
             pos_v, negout_v, sem):
    wid = lax.axis_index("s") * NC + lax.axis_index("c")

    def chunk_body(c, _):
        base = wid * ROWS_PER_W + c * CHUNK
        pltpu.sync_copy(tar_idx.at[pl.ds(base, CHUNK)], tar_i)
        pltpu.sync_copy(ctx_idx.at[pl.ds(base, CHUNK)], ctx_i)
        pltpu.sync_copy(char_num.at[pl.ds(base, CHUNK)], num_v)
        for j in range(NEG):
            pltpu.sync_copy(neg_idx.at[pl.ds(j * B + base, CHUNK)], neg_i.at[j])
        for j in range(MAXWL):
            pltpu.sync_copy(char_idx.at[pl.ds(j * B + base, CHUNK)],
                            char_i.at[j])
        cps = [
            pltpu.async_copy(emb0.at[tar_i], tar_r, sem),
            pltpu.async_copy(emb1.at[ctx_i], ctx_r, sem),
        ]
        for j in range(NEG):
            cps.append(pltpu.async_copy(emb1.at[neg_i.at[j]], neg_r.at[j], sem))
        for j in range(MAXWL):
            cps.append(pltpu.async_copy(emb0c.at[char_i.at[j]], char_r.at[j], sem))
        for cp in cps:
            cp.wait()

        def group_body(g, carry):
            gbase = g * LANES
            invv = 0.5 / num_v[pl.ds(gbase, LANES)]
            lane_iota = lax.iota(jnp.int32, LANES)
            posvec = jnp.zeros((LANES,), jnp.float32)
            negvecs = [jnp.zeros((LANES,), jnp.float32) for _ in range(NEG)]
            for l in range(LANES):
                r = gbase + l
                cs = [char_r[0, r, pl.ds(16 * k, 16)] for k in range(KV)]
                for j in range(1, MAXWL):
                    cs = [cs[k] + char_r[j, r, pl.ds(16 * k, 16)]
                          for k in range(KV)]
                inv = invv[l]
                avg = [tar_r[r, pl.ds(16 * k, 16)] * 0.5 + cs[k] * inv
                       for k in range(KV)]
                acc = avg[0] * ctx_r[r, pl.ds(0, 16)]
                for k in range(1, KV):
                    acc = acc + avg[k] * ctx_r[r, pl.ds(16 * k, 16)]
                sel = lane_iota == l
                posvec = jnp.where(sel, jnp.sum(acc), posvec)
                for j in range(NEG):
                    accn = avg[0] * neg_r[j, r, pl.ds(0, 16)]
                    for k in range(1, KV):
                        accn = accn + avg[k] * neg_r[j, r, pl.ds(16 * k, 16)]
                    negvecs[j] = jnp.where(sel, jnp.sum(accn), negvecs[j])
            pos_v[pl.ds(gbase, LANES)] = posvec
            for j in range(NEG):
                negout_v[j, pl.ds(gbase, LANES)] = negvecs[j]
            return carry

        lax.fori_loop(0, CHUNK // LANES, group_body, 0)

        pltpu.sync_copy(pos_v, pos_out.at[pl.ds(base, CHUNK)])
        for j in range(NEG):
            pltpu.sync_copy(negout_v.at[j],
                            neg_out.at[pl.ds(j * B + base, CHUNK)])
        return 0

    lax.fori_loop(0, NCHUNK, chunk_body, 0)


_sc_dots = pl.kernel(
    _sc_body,
    out_type=(
        jax.ShapeDtypeStruct((B,), jnp.float32),
        jax.ShapeDtypeStruct((NEG * B,), jnp.float32),
    ),
    mesh=plsc.VectorSubcoreMesh(core_axis_name="c", subcore_axis_name="s"),
    compiler_params=pltpu.CompilerParams(needs_layout_passes=False,
                                         use_tc_tiling_on_sc=False),
    scratch_types=[
        pltpu.VMEM((CHUNK,), jnp.int32),
        pltpu.VMEM((CHUNK,), jnp.int32),
        pltpu.VMEM((NEG, CHUNK), jnp.int32),
        pltpu.VMEM((MAXWL, CHUNK), jnp.int32),
        pltpu.VMEM((CHUNK,), jnp.float32),
        pltpu.VMEM((CHUNK, DIM), jnp.float32),
        pltpu.VMEM((CHUNK, DIM), jnp.float32),
        pltpu.VMEM((NEG, CHUNK, DIM), jnp.float32),
        pltpu.VMEM((MAXWL, CHUNK, DIM), jnp.float32),
        pltpu.VMEM((CHUNK,), jnp.float32),
        pltpu.VMEM((NEG, CHUNK), jnp.float32),
        pltpu.SemaphoreType.DMA,
    ],
)


def _loss_body(pos_ref, neg_ref, mask_ref, out_ref):
    p = jnp.clip(pos_ref[...], -10.0, 10.0)
    pos_loss = jnp.sum(jnp.log1p(jnp.exp(-p)))
    z = jnp.clip(-neg_ref[...], -10.0, 10.0)
    neg_loss = jnp.sum(jnp.log1p(jnp.exp(-z)) * mask_ref[...])
    out_ref[0, 0] = pos_loss + neg_loss


def _tc_loss(pos2, neg2, mask2):
    return pl.pallas_call(
        _loss_body,
        out_shape=jax.ShapeDtypeStruct((1, 1), jnp.float32),
        out_specs=pl.BlockSpec(memory_space=pltpu.SMEM),
    )(pos2, neg2, mask2)


@jax.jit
def kernel(word_data, char_data, emb0, emb1, emb0_char):
    tar_idx = word_data[:, 1]
    ctx_idx = word_data[:, 0]
    negT_idx = word_data[:, 2:2 + NEG].T.reshape(-1)
    maskT = word_data[:, 2 + NEG:].T.astype(jnp.float32)
    charT_idx = char_data[:, :MAXWL].T.reshape(-1)
    char_num = char_data[:, MAXWL].astype(jnp.float32)

    pos_ips, neg_ips = _sc_dots(tar_idx, ctx_idx, negT_idx, charT_idx,
                                char_num, emb0, emb1, emb0_char)

    loss = _tc_loss(pos_ips.reshape(B // 128, 128),
                    neg_ips.reshape(NEG * B // 128, 128),
                    maskT.reshape(NEG * B // 128, 128))
    return loss[0, 0]


# speedup vs baseline: 1.7713x; 1.0364x over previous
import jax
import jax.numpy as jnp
from jax import lax
from jax.experimental import pallas as pl
from jax.experimental.pallas import tpu as pltpu
from jax.experimental.pallas import tpu_sc as plsc

VOCAB = 1000000
CHAR_VOCAB = 20000
DIM = 64
B = 16384
NEG = 5
MAXWL = 8

NC = 2
NS = 16
NW = NC * NS
ROWS_PER_W = B // NW
CHUNK = 32
NCHUNK = ROWS_PER_W // CHUNK
LANES = 16
KV = DIM // LANES
NSEC = 2 + NEG + MAXWL


def _sc_body(idx_hbm, num_hbm,
             emb0, emb1, emb0c,
             pos_out, neg_out,
             idx_all, num_all, rows_r, pos_all, neg_all, sem0, sem1):
    wid = lax.axis_index("s") * NC + lax.axis_index("c")

    for s in range(NSEC):
        pltpu.sync_copy(idx_hbm.at[wid * NSEC + s], idx_all.at[s])
    pltpu.sync_copy(num_hbm.at[wid], num_all)

    tables = [emb0, emb1] + [emb1] * NEG + [emb0c] * MAXWL
    sems = (sem0, sem1)

    def fire(c, b):
        for s in range(NSEC):
            pltpu.async_copy(tables[s].at[idx_all.at[s, c]],
                             rows_r.at[b, s], sems[b])

    def drain(b):
        for s in range(NSEC):
            pltpu.make_async_copy(tables[s].at[idx_all.at[s, 0]],
                                  rows_r.at[b, s], sems[b]).wait()

    def compute(c, b):
        def group_body(g, carry):
            invv = 0.5 / num_all[c, pl.ds(g * LANES, LANES)]
            lane_iota = lax.iota(jnp.int32, LANES)
            posvec = jnp.zeros((LANES,), jnp.float32)
            negvecs = [jnp.zeros((LANES,), jnp.float32) for _ in range(NEG)]
            for l in range(LANES):
                r = g * LANES + l
                cs = [rows_r[b, 7, r, pl.ds(16 * k, 16)] for k in range(KV)]
                for j in range(1, MAXWL):
                    cs = [cs[k] + rows_r[b, 7 + j, r, pl.ds(16 * k, 16)]
                          for k in range(KV)]
                inv = invv[l]
                avg = [rows_r[b, 0, r, pl.ds(16 * k, 16)] * 0.5 + cs[k] * inv
                       for k in range(KV)]
                acc = avg[0] * rows_r[b, 1, r, pl.ds(0, 16)]
                for k in range(1, KV):
                    acc = acc + avg[k] * rows_r[b, 1, r, pl.ds(16 * k, 16)]
                sel = lane_iota == l
                posvec = jnp.where(sel, jnp.sum(acc), posvec)
                for j in range(NEG):
                    accn = avg[0] * rows_r[b, 2 + j, r, pl.ds(0, 16)]
                    for k in range(1, KV):
                        accn = accn + avg[k] * rows_r[b, 2 + j, r,
                                                      pl.ds(16 * k, 16)]
                    negvecs[j] = jnp.where(sel, jnp.sum(accn), negvecs[j])
            obase = c * CHUNK + g * LANES
            pos_all[pl.ds(obase, LANES)] = posvec
            for j in range(NEG):
                neg_all[j, pl.ds(obase, LANES)] = negvecs[j]
            return carry

        lax.fori_loop(0, CHUNK // LANES, group_body, 0)

    fire(0, 0)

    def body2(cc, _):
        c0 = cc * 2
        fire(c0 + 1, 1)
        drain(0)
        compute(c0, 0)

        @pl.when(c0 + 2 < NCHUNK)
        def _():
            fire(c0 + 2, 0)

        drain(1)
        compute(c0 + 1, 1)
        return 0

    lax.fori_loop(0, NCHUNK // 2, body2, 0)

    obase = wid * ROWS_PER_W
    pltpu.sync_copy(pos_all, pos_out.at[pl.ds(obase, ROWS_PER_W)])
    for j in range(NEG):
        pltpu.sync_copy(neg_all.at[j],
                        neg_out.at[pl.ds(j * B + obase, ROWS_PER_W)])


_sc_dots = pl.kernel(
    _sc_body,
    out_type=(
        jax.ShapeDtypeStruct((B,), jnp.float32),
        jax.ShapeDtypeStruct((NEG * B,), jnp.float32),
    ),
    mesh=plsc.VectorSubcoreMesh(core_axis_name="c", subcore_axis_name="s"),
    compiler_params=pltpu.CompilerParams(needs_layout_passes=False,
                                         use_tc_tiling_on_sc=False),
    scratch_types=[
        pltpu.VMEM((NSEC, NCHUNK, CHUNK), jnp.int32),
        pltpu.VMEM((NCHUNK, CHUNK), jnp.float32),
        pltpu.VMEM((2, NSEC, CHUNK, DIM), jnp.float32),
        pltpu.VMEM((ROWS_PER_W,), jnp.float32),
        pltpu.VMEM((NEG, ROWS_PER_W), jnp.float32),
        pltpu.SemaphoreType.DMA,
        pltpu.SemaphoreType.DMA,
    ],
)


def _loss_body(pos_ref, neg_ref, mask_ref, out_ref):
    p = jnp.clip(pos_ref[...], -10.0, 10.0)
    pos_loss = jnp.sum(jnp.log1p(jnp.exp(-p)))
    z = jnp.clip(-neg_ref[...], -10.0, 10.0)
    neg_loss = jnp.sum(jnp.log1p(jnp.exp(-z)) * mask_ref[...])
    out_ref[0, 0] = pos_loss + neg_loss


def _tc_loss(pos2, neg2, mask2):
    return pl.pallas_call(
        _loss_body,
        out_shape=jax.ShapeDtypeStruct((1, 1), jnp.float32),
        out_specs=pl.BlockSpec(memory_space=pltpu.SMEM),
    )(pos2, neg2, mask2)


@jax.jit
def kernel(word_data, char_data, emb0, emb1, emb0_char):
    idx_cols = jnp.concatenate(
        [word_data[:, jnp.array([1, 0, 2, 3, 4, 5, 6])],
         char_data[:, :MAXWL]], axis=1)
    idx_hbm = (idx_cols.T.reshape(NSEC, NW, ROWS_PER_W)
               .transpose(1, 0, 2).reshape(NW * NSEC, NCHUNK, CHUNK))
    num_hbm = (char_data[:, MAXWL].astype(jnp.float32)
               .reshape(NW, NCHUNK, CHUNK))
    maskT = word_data[:, 2 + NEG:].T.astype(jnp.float32)

    pos_ips, neg_ips = _sc_dots(idx_hbm, num_hbm, emb0, emb1, emb0_char)

    loss = _tc_loss(pos_ips.reshape(B // 128, 128),
                    neg_ips.reshape(NEG * B // 128, 128),
                    maskT.reshape(NEG * B // 128, 128))
    return loss[0, 0]
